# baseline (speedup 1.0000x reference)
import jax
import jax.numpy as jnp
from jax import lax
from jax.experimental import pallas as pl
from jax.experimental.pallas import tpu as pltpu
from jax.experimental.pallas import tpu_sc as plsc

EMB = 64
_NC = 2
_NS = 16
NW = _NC * _NS
G = 128
R = 2
CROWS = 1600


def _concat_body(w_ref, p_ref, comb_ref):
    comb_ref[:, : EMB] = w_ref[...]
    comb_ref[:, EMB:] = p_ref[...]


def _concat_tc(word_table, pe_table):
    v = word_table.shape[0]
    return pl.pallas_call(
        _concat_body,
        grid=(v // CROWS,),
        in_specs=[
            pl.BlockSpec((CROWS, EMB), lambda i: (i, 0)),
            pl.BlockSpec((CROWS, EMB), lambda i: (i, 0)),
        ],
        out_specs=pl.BlockSpec((CROWS, 2 * EMB), lambda i: (i, 0)),
        out_shape=jax.ShapeDtypeStruct((v, 2 * EMB), jnp.float32),
    )(word_table, pe_table)


def _emb_body(x_hbm, comb_hbm, out_hbm, idx_v, gbufs, obufs, sems_g, sems_o):
    ng = x_hbm.shape[0] // NW
    wid = lax.axis_index("s") * _NC + lax.axis_index("c")
    pltpu.sync_copy(x_hbm.at[pl.ds(wid * ng, ng)], idx_v)
    base = wid * ng * G

    def fire(g, k):
        pltpu.async_copy(comb_hbm.at[idx_v.at[g]], gbufs[k], sems_g[k])

    def wait_gather(k):
        pltpu.make_async_copy(comb_hbm.at[idx_v.at[0]], gbufs[k], sems_g[k]).wait()

    def drain_out(k):
        pltpu.make_async_copy(obufs[k], out_hbm.at[pl.ds(base, G)], sems_o[k]).wait()

    for k in range(R):
        fire(k, k)

    @pl.loop(0, ng, step=R)
    def _pair(g):
        for k in range(R):
            gi = g + k
            wait_gather(k)

            @pl.loop(0, G, unroll=4)
            def _row(j):
                for c in range(EMB // 16):
                    s = pl.ds(c * 16, 16)
                    obufs[k][j, s] = gbufs[k][j, s] + gbufs[k][j, pl.ds(EMB + c * 16, 16)]

            @pl.when(gi + R < ng)
            def _():
                fire(gi + R, k)

            @pl.when(gi >= R)
            def _():
                drain_out(k)

            pltpu.async_copy(obufs[k], out_hbm.at[pl.ds(base + gi * G, G)], sems_o[k])

    for k in range(R):
        drain_out(k)


def kernel(x, word_table, pe_table):
    b, s = x.shape
    n = b * s
    xg = x.reshape(n // G, G)
    comb = _concat_tc(word_table, pe_table)
    mesh = plsc.VectorSubcoreMesh(core_axis_name="c", subcore_axis_name="s")
    out = pl.kernel(
        _emb_body,
        out_type=jax.ShapeDtypeStruct((n, EMB), jnp.float32),
        mesh=mesh,
        scratch_types=[
            pltpu.VMEM((n // G // NW, G), jnp.int32),
            [pltpu.VMEM((G, 2 * EMB), jnp.float32) for _ in range(R)],
            [pltpu.VMEM((G, EMB), jnp.float32) for _ in range(R)],
            [pltpu.SemaphoreType.DMA for _ in range(R)],
            [pltpu.SemaphoreType.DMA for _ in range(R)],
        ],
    )(xg, comb)
    return out.reshape(b, s, EMB)


# speedup vs baseline: 1.5967x; 1.5967x over previous
import jax
import jax.numpy as jnp
from jax import lax
from jax.experimental import pallas as pl
from jax.experimental.pallas import tpu as pltpu
from jax.experimental.pallas import tpu_sc as plsc

EMB = 64
_NC = 2
_NS = 16
NW = _NC * _NS
G = 128
R = 2


def _emb_body(x_hbm, sum_hbm, out_hbm, idx_v, gbufs, obufs, sems_g, sems_o):
    ng = x_hbm.shape[0] // NW
    wid = lax.axis_index("s") * _NC + lax.axis_index("c")
    pltpu.sync_copy(x_hbm.at[pl.ds(wid * ng, ng)], idx_v)
    base = wid * ng * G

    def fire(g, k):
        pltpu.async_copy(sum_hbm.at[idx_v.at[g]], gbufs[k], sems_g[k])

    def wait_gather(k):
        pltpu.make_async_copy(sum_hbm.at[idx_v.at[0]], gbufs[k], sems_g[k]).wait()

    def drain_out(k):
        pltpu.make_async_copy(obufs[k], out_hbm.at[pl.ds(base, G)], sems_o[k]).wait()

    for k in range(R):
        fire(k, k)

    @pl.loop(0, ng, step=R)
    def _pair(g):
        for k in range(R):
            gi = g + k
            wait_gather(k)

            @pl.loop(0, G, unroll=4)
            def _row(j):
                for c in range(EMB // 16):
                    s = pl.ds(c * 16, 16)
                    obufs[k][j, s] = gbufs[k][j, s]

            @pl.when(gi + R < ng)
            def _():
                fire(gi + R, k)

            @pl.when(gi >= R)
            def _():
                drain_out(k)

            pltpu.async_copy(obufs[k], out_hbm.at[pl.ds(base + gi * G, G)], sems_o[k])

    for k in range(R):
        drain_out(k)


def kernel(x, word_table, pe_table):
    b, s = x.shape
    n = b * s
    xg = x.reshape(n // G, G)
    sum128 = jnp.pad(word_table + pe_table, ((0, 0), (0, EMB)))
    mesh = plsc.VectorSubcoreMesh(core_axis_name="c", subcore_axis_name="s")
    out = pl.kernel(
        _emb_body,
        out_type=jax.ShapeDtypeStruct((n, EMB), jnp.float32),
        mesh=mesh,
        scratch_types=[
            pltpu.VMEM((n // G // NW, G), jnp.int32),
            [pltpu.VMEM((G, 2 * EMB), jnp.float32) for _ in range(R)],
            [pltpu.VMEM((G, EMB), jnp.float32) for _ in range(R)],
            [pltpu.SemaphoreType.DMA for _ in range(R)],
            [pltpu.SemaphoreType.DMA for _ in range(R)],
        ],
    )(xg, sum128)
    return out.reshape(b, s, EMB)
